# Initial kernel scaffold; baseline (speedup 1.0000x reference)
import jax
import jax.numpy as jnp
from jax.experimental import pallas as pl


def kernel(x, edge_index, batch, W1l, b1l, W1r, W2l, b2l, W2r, Wa1, ba1, Wa2, ba2, Wc1, bc1, Wc2, bc2):
    raise NotImplementedError("write your pallas kernel here")


# speedup vs baseline: 7.0676x; 7.0676x over previous
import functools

import jax
import jax.numpy as jnp
from jax import lax
from jax.experimental import pallas as pl
from jax.experimental.pallas import tpu as pltpu
from jax.experimental.pallas import tpu_sc as plsc

N = 10000
F = 128
H = 64
G = 64

N_PAD = 10240
BLK = 512
CHUNK = 128
CW = 16
NC = 2
NS = 16
NW = NC * NS
RPT = N_PAD // NS


def _seg_body_counts(nchunks, p_hbm, src_hbm, dst_hbm, z64, z16, ones_hbm,
                     out_acc, out_cnt,
                     idx_s, idx_d, rows_v, ones_v, acc_sh, cnt_sh, sem):
    cid = lax.axis_index("c")
    sid = lax.axis_index("s")
    wid = sid * NC + cid
    base = sid * RPT
    pltpu.sync_copy(z64.at[pl.ds(base, RPT)], acc_sh.at[pl.ds(base, RPT)])
    pltpu.sync_copy(z16.at[pl.ds(base, RPT)], cnt_sh.at[pl.ds(base, RPT)])
    pltpu.sync_copy(src_hbm.at[wid], idx_s)
    pltpu.sync_copy(dst_hbm.at[wid], idx_d)
    pltpu.sync_copy(ones_hbm, ones_v)
    plsc.subcore_barrier()

    def step(j, carry):
        pltpu.async_copy(p_hbm.at[idx_s.at[j]], rows_v, sem).wait()
        pltpu.sync_copy(rows_v, acc_sh.at[idx_d.at[j]], add=True)
        pltpu.sync_copy(ones_v, cnt_sh.at[idx_d.at[j]], add=True)
        return carry

    lax.fori_loop(0, nchunks, step, 0)
    plsc.subcore_barrier()
    pltpu.sync_copy(acc_sh.at[pl.ds(base, RPT)], out_acc.at[cid, pl.ds(base, RPT)])
    pltpu.sync_copy(cnt_sh.at[pl.ds(base, RPT)], out_cnt.at[cid, pl.ds(base, RPT)])


def _seg_body_nocnt(nchunks, p_hbm, src_hbm, dst_hbm, z64,
                    out_acc,
                    idx_s, idx_d, rows_v, acc_sh, sem):
    cid = lax.axis_index("c")
    sid = lax.axis_index("s")
    wid = sid * NC + cid
    base = sid * RPT
    pltpu.sync_copy(z64.at[pl.ds(base, RPT)], acc_sh.at[pl.ds(base, RPT)])
    pltpu.sync_copy(src_hbm.at[wid], idx_s)
    pltpu.sync_copy(dst_hbm.at[wid], idx_d)
    plsc.subcore_barrier()

    def step(j, carry):
        pltpu.async_copy(p_hbm.at[idx_s.at[j]], rows_v, sem).wait()
        pltpu.sync_copy(rows_v, acc_sh.at[idx_d.at[j]], add=True)
        return carry

    lax.fori_loop(0, nchunks, step, 0)
    plsc.subcore_barrier()
    pltpu.sync_copy(acc_sh.at[pl.ds(base, RPT)], out_acc.at[cid, pl.ds(base, RPT)])


def _seg_sum_counts(p, src3, dst3, z64, z16, ones, nchunks):
    mesh = plsc.VectorSubcoreMesh(core_axis_name="c", subcore_axis_name="s")
    kern = pl.kernel(
        functools.partial(_seg_body_counts, nchunks),
        mesh=mesh,
        compiler_params=pltpu.CompilerParams(use_tc_tiling_on_sc=False),
        out_type=[
            jax.ShapeDtypeStruct((NC, N_PAD, H), jnp.float32),
            jax.ShapeDtypeStruct((NC, N_PAD, CW), jnp.float32),
        ],
        scratch_types=[
            pltpu.VMEM((nchunks, CHUNK), jnp.int32),
            pltpu.VMEM((nchunks, CHUNK), jnp.int32),
            pltpu.VMEM((CHUNK, H), jnp.float32),
            pltpu.VMEM((CHUNK, CW), jnp.float32),
            pltpu.VMEM_SHARED((N_PAD, H), jnp.float32),
            pltpu.VMEM_SHARED((N_PAD, CW), jnp.float32),
            pltpu.SemaphoreType.DMA,
        ],
    )
    return kern(p, src3, dst3, z64, z16, ones)


def _seg_sum_nocnt(p, src3, dst3, z64, nchunks):
    mesh = plsc.VectorSubcoreMesh(core_axis_name="c", subcore_axis_name="s")
    kern = pl.kernel(
        functools.partial(_seg_body_nocnt, nchunks),
        mesh=mesh,
        compiler_params=pltpu.CompilerParams(use_tc_tiling_on_sc=False),
        out_type=jax.ShapeDtypeStruct((NC, N_PAD, H), jnp.float32),
        scratch_types=[
            pltpu.VMEM((nchunks, CHUNK), jnp.int32),
            pltpu.VMEM((nchunks, CHUNK), jnp.int32),
            pltpu.VMEM((CHUNK, H), jnp.float32),
            pltpu.VMEM_SHARED((N_PAD, H), jnp.float32),
            pltpu.SemaphoreType.DMA,
        ],
    )
    return kern(p, src3, dst3, z64)


def _proj_body(x_ref, wl_ref, wr_ref, p_ref, r_ref):
    xb = x_ref[...]
    p_ref[...] = jnp.dot(xb, wl_ref[...], preferred_element_type=jnp.float32)
    r_ref[...] = jnp.dot(xb, wr_ref[...], preferred_element_type=jnp.float32)


def _project(x_p, wlT, wrT):
    return pl.pallas_call(
        _proj_body,
        grid=(N_PAD // BLK,),
        in_specs=[
            pl.BlockSpec((BLK, F), lambda i: (i, 0)),
            pl.BlockSpec((F, H), lambda i: (0, 0)),
            pl.BlockSpec((F, H), lambda i: (0, 0)),
        ],
        out_specs=[
            pl.BlockSpec((BLK, H), lambda i: (i, 0)),
            pl.BlockSpec((BLK, H), lambda i: (i, 0)),
        ],
        out_shape=[
            jax.ShapeDtypeStruct((N_PAD, H), jnp.float32),
            jax.ShapeDtypeStruct((N_PAD, H), jnp.float32),
        ],
    )(x_p, wlT, wrT)


def _combine_body(acc_ref, cnt_ref, r_ref, b_ref, wl_ref, wr_ref, p2_ref, r2_ref):
    a = acc_ref[0] + acc_ref[1]
    c = cnt_ref[0, :, :1] + cnt_ref[1, :, :1]
    h = jnp.maximum(a / jnp.maximum(c, 1.0) + b_ref[...] + r_ref[...], 0.0)
    p2_ref[...] = jnp.dot(h, wl_ref[...], preferred_element_type=jnp.float32)
    r2_ref[...] = jnp.dot(h, wr_ref[...], preferred_element_type=jnp.float32)


def _combine(acc, cnt, r1, b2d, wlT, wrT):
    return pl.pallas_call(
        _combine_body,
        grid=(N_PAD // BLK,),
        in_specs=[
            pl.BlockSpec((NC, BLK, H), lambda i: (0, i, 0)),
            pl.BlockSpec((NC, BLK, CW), lambda i: (0, i, 0)),
            pl.BlockSpec((BLK, H), lambda i: (i, 0)),
            pl.BlockSpec((1, H), lambda i: (0, 0)),
            pl.BlockSpec((H, H), lambda i: (0, 0)),
            pl.BlockSpec((H, H), lambda i: (0, 0)),
        ],
        out_specs=[
            pl.BlockSpec((BLK, H), lambda i: (i, 0)),
            pl.BlockSpec((BLK, H), lambda i: (i, 0)),
        ],
        out_shape=[
            jax.ShapeDtypeStruct((N_PAD, H), jnp.float32),
            jax.ShapeDtypeStruct((N_PAD, H), jnp.float32),
        ],
    )(acc, cnt, r1, b2d, wlT, wrT)


def _final_body(acc_ref, cnt_ref, r_ref, b_ref, batch_ref,
                wa1_ref, ba1_ref, wa2_ref, ba2_ref,
                wc1_ref, bc1_ref, wc2_ref, bc2_ref,
                mu_ref, val_ref, sums_ref, cnts_ref):
    i = pl.program_id(0)

    @pl.when(i == 0)
    def _():
        sums_ref[...] = jnp.zeros_like(sums_ref)
        cnts_ref[...] = jnp.zeros_like(cnts_ref)

    a = acc_ref[0] + acc_ref[1]
    c = cnt_ref[0, :, :1] + cnt_ref[1, :, :1]
    h = jnp.maximum(a / jnp.maximum(c, 1.0) + b_ref[...] + r_ref[...], 0.0)
    oh = (batch_ref[...] == lax.broadcasted_iota(jnp.int32, (G, BLK), 0)
          ).astype(jnp.float32)
    sums_ref[...] += jnp.dot(oh, h, preferred_element_type=jnp.float32)
    cnts_ref[...] += jnp.sum(oh, axis=1, keepdims=True)

    @pl.when(i == pl.num_programs(0) - 1)
    def _():
        pooled = sums_ref[...] / jnp.maximum(cnts_ref[...], 1.0)
        ha = jnp.maximum(
            jnp.dot(pooled, wa1_ref[...], preferred_element_type=jnp.float32)
            + ba1_ref[...], 0.0)
        mu_ref[...] = (jnp.dot(ha, wa2_ref[...], preferred_element_type=jnp.float32)
                       + ba2_ref[...])
        hc = jnp.maximum(
            jnp.dot(pooled, wc1_ref[...], preferred_element_type=jnp.float32)
            + bc1_ref[...], 0.0)
        val_ref[...] = (jnp.dot(hc, wc2_ref[...], preferred_element_type=jnp.float32)
                        + bc2_ref[...])


def _final(acc, cnt, r2, b2d, batch_row, wa1T, ba1, wa2T, ba2, wc1T, bc1, wc2T, bc2):
    A = wa2T.shape[1]
    const = lambda shape: pl.BlockSpec(shape, lambda i: tuple(0 for _ in shape))
    return pl.pallas_call(
        _final_body,
        grid=(N_PAD // BLK,),
        in_specs=[
            pl.BlockSpec((NC, BLK, H), lambda i: (0, i, 0)),
            pl.BlockSpec((NC, BLK, CW), lambda i: (0, i, 0)),
            pl.BlockSpec((BLK, H), lambda i: (i, 0)),
            const((1, H)),
            pl.BlockSpec((1, BLK), lambda i: (0, i)),
            const((H, H)), const((1, H)), const((H, A)), const((1, A)),
            const((H, H)), const((1, H)), const((H, 1)), const((1, 1)),
        ],
        out_specs=[
            pl.BlockSpec((G, A), lambda i: (0, 0)),
            pl.BlockSpec((G, 1), lambda i: (0, 0)),
        ],
        out_shape=[
            jax.ShapeDtypeStruct((G, A), jnp.float32),
            jax.ShapeDtypeStruct((G, 1), jnp.float32),
        ],
        scratch_shapes=[
            pltpu.VMEM((G, H), jnp.float32),
            pltpu.VMEM((G, 1), jnp.float32),
        ],
    )(acc, cnt, r2, b2d, batch_row, wa1T, ba1, wa2T, ba2, wc1T, bc1, wc2T, bc2)


def kernel(x, edge_index, batch, W1l, b1l, W1r, W2l, b2l, W2r,
           Wa1, ba1, Wa2, ba2, Wc1, bc1, Wc2, bc2):
    E = edge_index.shape[1]
    nchunks = -(-E // (NW * CHUNK))
    e_pad = NW * nchunks * CHUNK

    x_p = jnp.pad(x, ((0, N_PAD - N), (0, 0)))
    pad = jnp.full((e_pad - E,), N, jnp.int32)
    src3 = jnp.concatenate([edge_index[0], pad]).reshape(NW, nchunks, CHUNK)
    dst3 = jnp.concatenate([edge_index[1], pad]).reshape(NW, nchunks, CHUNK)
    batch_row = jnp.pad(batch, (0, N_PAD - N), constant_values=G).reshape(1, N_PAD)

    z64 = jnp.zeros((N_PAD, H), jnp.float32)
    z16 = jnp.zeros((N_PAD, CW), jnp.float32)
    ones = jnp.ones((CHUNK, CW), jnp.float32)

    p1, r1 = _project(x_p, W1l.T, W1r.T)
    acc1, cnt = _seg_sum_counts(p1, src3, dst3, z64, z16, ones, nchunks)
    p2, r2 = _combine(acc1, cnt, r1, b1l.reshape(1, H), W2l.T, W2r.T)
    acc2 = _seg_sum_nocnt(p2, src3, dst3, z64, nchunks)
    mu, value = _final(acc2, cnt, r2, b2l.reshape(1, H), batch_row,
                       Wa1.T, ba1.reshape(1, -1), Wa2.T, ba2.reshape(1, -1),
                       Wc1.T, bc1.reshape(1, -1), Wc2.T, bc2.reshape(1, -1))
    return (mu, value)
